# baseline (speedup 1.0000x reference)
import jax
import jax.numpy as jnp
from jax import lax
from jax.experimental import pallas as pl
from jax.experimental.pallas import tpu as pltpu
from jax.experimental.pallas import tpu_sc as plsc

B = 4
S = 8192
D = 1024
L = 16
NC = 2
NS = 16
NW = NC * NS
S_PER_W = S // NW
K = 8
NCHUNK = S_PER_W // K
NTRIPLE = (NCHUNK - 2) // 3
SEG = 4
SEGV = D // L // SEG


def _body(ids_hbm, tok_hbm, pos_hbm, out_hbm,
          idx_v, buf0, buf1, buf2, pos0, pos1, pos2,
          gsem0, gsem1, gsem2, osem0, osem1, osem2,
          psem0, psem1, psem2, isem0, isem1, isem2):
    wid = lax.axis_index("s") * NC + lax.axis_index("c")
    base = wid * S_PER_W
    bufs = (buf0, buf1, buf2)
    poss = (pos0, pos1, pos2)
    gsems = (gsem0, gsem1, gsem2)
    osems = (osem0, osem1, osem2)
    psems = (psem0, psem1, psem2)
    isems = (isem0, isem1, isem2)

    def add_chunk(p):
        buf = bufs[p]
        pos_ref = poss[p]

        @plsc.parallel_loop(0, K * SEG)
        def _(i):
            r = i // SEG
            c0 = (i % SEG) * (SEGV * L)
            for l in range(SEGV):
                sl = pl.ds(c0 + l * L, L)
                pv = pos_ref[r, sl]
                for b in range(B):
                    plsc.addupdate(buf.at[b * K + r, sl], pv)

    def issue_gather(p):
        pltpu.async_copy(tok_hbm.at[idx_v.at[p]], bufs[p], gsems[p])

    def wait_gather(p):
        pltpu.make_async_copy(tok_hbm.at[idx_v.at[0]], bufs[p],
                              gsems[p]).wait()

    def issue_writes(p, off):
        for b in range(B):
            pltpu.async_copy(bufs[p].at[pl.ds(b * K, K)],
                             out_hbm.at[b, pl.ds(off, K)], osems[p])

    def wait_writes(p):
        for b in range(B):
            pltpu.make_async_copy(bufs[p].at[pl.ds(b * K, K)],
                                  out_hbm.at[0, pl.ds(0, K)],
                                  osems[p]).wait()

    def prefetch(p, off):
        pltpu.async_copy(pos_hbm.at[pl.ds(off, K)], poss[p], psems[p])
        for b in range(B):
            pltpu.async_copy(ids_hbm.at[b, pl.ds(off, K)],
                             idx_v.at[p, pl.ds(b * K, K)], isems[p])

    def wait_pos(p):
        pltpu.make_async_copy(pos_hbm.at[pl.ds(0, K)], poss[p],
                              psems[p]).wait()

    def wait_idx(p):
        for b in range(B):
            pltpu.make_async_copy(ids_hbm.at[0, pl.ds(0, K)],
                                  idx_v.at[p, pl.ds(b * K, K)],
                                  isems[p]).wait()

    def chunk_body(j_dyn, p, pn, pf, first_pos_primed, skip_drain,
                   issue_next, do_prefetch):
        off = base + j_dyn * K
        wait_gather(p)
        if first_pos_primed is None:
            wait_pos(p)
        else:
            @pl.when(first_pos_primed)
            def _():
                wait_pos(p)
        if issue_next:
            if skip_drain is None:
                wait_writes(pn)
            else:
                @pl.when(skip_drain)
                def _():
                    wait_writes(pn)
            wait_idx(pn)
            issue_gather(pn)
        add_chunk(p)
        issue_writes(p, off)
        if do_prefetch:
            prefetch(pf, off + 2 * K)

    for b in range(B):
        pltpu.sync_copy(ids_hbm.at[b, pl.ds(base, K)],
                        idx_v.at[0, pl.ds(b * K, K)])
    pltpu.sync_copy(pos_hbm.at[pl.ds(base, K)], pos0)
    prefetch(1, base + K)
    issue_gather(0)

    def triple(m, carry):
        for jj in range(3):
            j = 3 * m + jj
            p, pn, pf = jj, (jj + 1) % 3, (jj + 2) % 3
            pos_guard = (m > 0) if jj == 0 else None
            drain_guard = (m > 0) if jj < 2 else None
            chunk_body(j, p, pn, pf, pos_guard, drain_guard,
                       issue_next=True, do_prefetch=True)
        return carry

    lax.fori_loop(0, NTRIPLE, triple, 0)
    j30 = 3 * NTRIPLE
    chunk_body(j30, 0, 1, 2, None, None, issue_next=True, do_prefetch=False)
    chunk_body(j30 + 1, 1, 2, 0, None, None, issue_next=False,
               do_prefetch=False)
    wait_writes(2)
    wait_writes(0)
    wait_writes(1)


def kernel(input_ids, tok_table, pos_table):
    mesh = plsc.VectorSubcoreMesh(core_axis_name="c", subcore_axis_name="s")
    k = pl.kernel(
        _body,
        out_type=jax.ShapeDtypeStruct((B, S, D), jnp.float32),
        mesh=mesh,
        scratch_types=[
            pltpu.VMEM((3, B * K), jnp.int32),
            pltpu.VMEM((B * K, D), jnp.float32),
            pltpu.VMEM((B * K, D), jnp.float32),
            pltpu.VMEM((B * K, D), jnp.float32),
            pltpu.VMEM((K, D), jnp.float32),
            pltpu.VMEM((K, D), jnp.float32),
            pltpu.VMEM((K, D), jnp.float32),
            pltpu.SemaphoreType.DMA,
            pltpu.SemaphoreType.DMA,
            pltpu.SemaphoreType.DMA,
            pltpu.SemaphoreType.DMA,
            pltpu.SemaphoreType.DMA,
            pltpu.SemaphoreType.DMA,
            pltpu.SemaphoreType.DMA,
            pltpu.SemaphoreType.DMA,
            pltpu.SemaphoreType.DMA,
            pltpu.SemaphoreType.DMA,
            pltpu.SemaphoreType.DMA,
            pltpu.SemaphoreType.DMA,
        ],
    )
    return k(input_ids, tok_table, pos_table)
